# baseline (speedup 1.0000x reference)
import functools

import jax
import jax.numpy as jnp
from jax import lax
from jax.experimental import pallas as pl
from jax.experimental.pallas import tpu as pltpu
from jax.experimental.pallas import tpu_sc as plsc


def _sc_gather_coeffs(t, sqrt_ac, sqrt_omac):
    B = t.shape[0]
    mesh = plsc.VectorSubcoreMesh(core_axis_name="c", subcore_axis_name="s")

    @functools.partial(
        pl.kernel,
        mesh=mesh,
        out_type=jax.ShapeDtypeStruct((2, B), jnp.float32),
        scratch_types=[
            pltpu.VMEM((B,), jnp.int32),
            pltpu.VMEM((B,), jnp.float32),
            pltpu.SemaphoreType.DMA,
        ],
        compiler_params=pltpu.CompilerParams(skip_device_barrier=True),
    )
    def gather_kernel(t_hbm, ac_hbm, omac_hbm, out_hbm, idx_v, val_v, sem):
        cid = lax.axis_index("c")
        sid = lax.axis_index("s")

        @pl.when(jnp.logical_and(cid == 0, sid == 0))
        def _():
            pltpu.sync_copy(t_hbm, idx_v)
            pltpu.async_copy(ac_hbm.at[idx_v], val_v, sem).wait()
            pltpu.sync_copy(val_v, out_hbm.at[0])

        @pl.when(jnp.logical_and(cid == 0, sid == 1))
        def _():
            pltpu.sync_copy(t_hbm, idx_v)
            pltpu.async_copy(omac_hbm.at[idx_v], val_v, sem).wait()
            pltpu.sync_copy(val_v, out_hbm.at[1])

    return gather_kernel(t, sqrt_ac, sqrt_omac)


def _tc_combine_head(xT, nT, t2, acp, omp, ka, rb):
    Rtot, B = xT.shape
    T = acp.shape[1]

    def body(t_ref, ac_ref, om_ref, x_ref, n_ref, o_ref, c_ref):
        @pl.when(pl.program_id(0) == 0)
        def _():
            rows = lax.broadcasted_iota(jnp.int32, (T, B), 0)
            onehot = jnp.where(rows == t_ref[...], 1.0, 0.0)
            c_ref[0:1, :] = lax.dot_general(
                ac_ref[...],
                onehot,
                (((1,), (0,)), ((), ())),
                precision=lax.Precision.HIGHEST,
            )
            c_ref[1:2, :] = lax.dot_general(
                om_ref[...],
                onehot,
                (((1,), (0,)), ((), ())),
                precision=lax.Precision.HIGHEST,
            )

        o_ref[...] = c_ref[0:1, :] * x_ref[...] + c_ref[1:2, :] * n_ref[...]

    return pl.pallas_call(
        body,
        grid=(ka // rb,),
        in_specs=[
            pl.BlockSpec((1, B), lambda i: (0, 0)),
            pl.BlockSpec((1, T), lambda i: (0, 0)),
            pl.BlockSpec((1, T), lambda i: (0, 0)),
            pl.BlockSpec((rb, B), lambda i: (i, 0)),
            pl.BlockSpec((rb, B), lambda i: (i, 0)),
        ],
        out_specs=pl.BlockSpec((rb, B), lambda i: (i, 0)),
        out_shape=jax.ShapeDtypeStruct((Rtot, B), jnp.float32),
        scratch_shapes=[pltpu.VMEM((2, B), jnp.float32)],
    )(t2, acp, omp, xT, nT)


def _tc_combine_tail(xT, nT, coeffs, partial, ka, rb):
    Rtot, B = xT.shape
    off = ka // rb

    def body(c_ref, x_ref, n_ref, p_ref, o_ref):
        o_ref[...] = c_ref[0:1, :] * x_ref[...] + c_ref[1:2, :] * n_ref[...]

    return pl.pallas_call(
        body,
        grid=((Rtot - ka) // rb,),
        in_specs=[
            pl.BlockSpec((2, B), lambda i: (0, 0)),
            pl.BlockSpec((rb, B), lambda i: (i + off, 0)),
            pl.BlockSpec((rb, B), lambda i: (i + off, 0)),
            pl.BlockSpec(memory_space=pl.ANY),
        ],
        out_specs=pl.BlockSpec((rb, B), lambda i: (i + off, 0)),
        out_shape=jax.ShapeDtypeStruct((Rtot, B), jnp.float32),
        input_output_aliases={3: 0},
    )(coeffs, xT, nT, partial)


def kernel(x_start, t, noise, sqrt_alphas_cumprod, sqrt_one_minus_alphas_cumprod):
    t32 = t.astype(jnp.int32)
    coeffs = _sc_gather_coeffs(
        t32, sqrt_alphas_cumprod, sqrt_one_minus_alphas_cumprod
    )
    B = x_start.shape[0]
    xT = jnp.transpose(x_start, (1, 2, 3, 0)).reshape(-1, B)
    nT = jnp.transpose(noise, (1, 2, 3, 0)).reshape(-1, B)
    rb = 9408
    ka = 4 * rb
    tpad = 1024
    acp = jnp.pad(sqrt_alphas_cumprod, (0, tpad - sqrt_alphas_cumprod.shape[0]))
    omp = jnp.pad(
        sqrt_one_minus_alphas_cumprod,
        (0, tpad - sqrt_one_minus_alphas_cumprod.shape[0]),
    )
    outA = _tc_combine_head(
        xT,
        nT,
        t32.reshape(1, B),
        acp.reshape(1, tpad),
        omp.reshape(1, tpad),
        ka,
        rb,
    )
    outT = _tc_combine_tail(xT, nT, coeffs, outA, ka, rb)
    out = outT.reshape(x_start.shape[1:] + (B,)).transpose(3, 0, 1, 2)
    return out


# speedup vs baseline: 1.0147x; 1.0147x over previous
import functools

import jax
import jax.numpy as jnp
from jax import lax
from jax.experimental import pallas as pl
from jax.experimental.pallas import tpu as pltpu
from jax.experimental.pallas import tpu_sc as plsc


def _sc_gather_coeffs(t, sqrt_ac, sqrt_omac):
    B = t.shape[0]
    mesh = plsc.VectorSubcoreMesh(core_axis_name="c", subcore_axis_name="s")

    @functools.partial(
        pl.kernel,
        mesh=mesh,
        out_type=[
            jax.ShapeDtypeStruct((B,), jnp.float32),
            jax.ShapeDtypeStruct((B,), jnp.float32),
        ],
        scratch_types=[
            pltpu.VMEM((B,), jnp.int32),
            pltpu.VMEM((B,), jnp.float32),
            pltpu.VMEM((B,), jnp.float32),
            pltpu.SemaphoreType.DMA,
        ],
    )
    def gather_kernel(t_hbm, ac_hbm, omac_hbm, s_hbm, sm_hbm, idx_v, s_v, sm_v, sem):
        cid = lax.axis_index("c")
        sid = lax.axis_index("s")

        @pl.when(jnp.logical_and(cid == 0, sid == 0))
        def _():
            pltpu.sync_copy(t_hbm, idx_v)
            pltpu.async_copy(ac_hbm.at[idx_v], s_v, sem).wait()
            pltpu.async_copy(omac_hbm.at[idx_v], sm_v, sem).wait()
            pltpu.sync_copy(s_v, s_hbm)
            pltpu.sync_copy(sm_v, sm_hbm)

    return gather_kernel(t, sqrt_ac, sqrt_omac)


def _tc_combine(xT, nT, s2, sm2, rb):
    Rtot, B = xT.shape

    def body(s_ref, sm_ref, x_ref, n_ref, o_ref):
        o_ref[...] = s_ref[...] * x_ref[...] + sm_ref[...] * n_ref[...]

    return pl.pallas_call(
        body,
        grid=(Rtot // rb,),
        in_specs=[
            pl.BlockSpec((1, B), lambda i: (0, 0)),
            pl.BlockSpec((1, B), lambda i: (0, 0)),
            pl.BlockSpec((rb, B), lambda i: (i, 0)),
            pl.BlockSpec((rb, B), lambda i: (i, 0)),
        ],
        out_specs=pl.BlockSpec((rb, B), lambda i: (i, 0)),
        out_shape=jax.ShapeDtypeStruct((Rtot, B), jnp.float32),
    )(s2, sm2, xT, nT)


def kernel(x_start, t, noise, sqrt_alphas_cumprod, sqrt_one_minus_alphas_cumprod):
    s, sm = _sc_gather_coeffs(
        t.astype(jnp.int32), sqrt_alphas_cumprod, sqrt_one_minus_alphas_cumprod
    )
    B = x_start.shape[0]
    xT = jnp.transpose(x_start, (1, 2, 3, 0)).reshape(-1, B)
    nT = jnp.transpose(noise, (1, 2, 3, 0)).reshape(-1, B)
    outT = _tc_combine(xT, nT, s.reshape(1, B), sm.reshape(1, B), rb=9408)
    out = outT.reshape(x_start.shape[1:] + (B,)).transpose(3, 0, 1, 2)
    return out
